# baseline (speedup 1.0000x reference)
import functools

import jax
import jax.numpy as jnp
from jax import lax
from jax.experimental import pallas as pl
from jax.experimental.pallas import tpu as pltpu
from jax.experimental.pallas import tpu_sc as plsc

VOCAB = 100000
EMBED_DIM = 64
BATCH = 16384
HIST = 50

NUM_CORES = 2
NUM_SUBCORES = 16
NUM_TILES = NUM_CORES * NUM_SUBCORES
LANES = 16
VPR = EMBED_DIM // LANES

BAGS_PER_TILE = BATCH // NUM_TILES
CHUNK_BAGS = 16
IDX_PER_CHUNK = CHUNK_BAGS * HIST
N_CHUNKS = BAGS_PER_TILE // CHUNK_BAGS
IDS_PAD = 64
HIST_GATHER = 56
PAD_PER_BAG = HIST_GATHER - HIST
ROWS_PER_CHUNK = CHUNK_BAGS * HIST_GATHER


def _sc_body(ids_hbm, table_hbm, out_hbm,
             idx0, idx1, rows0, rows1, out0, out1, row0_v,
             gsem0, gsem1, osem0, osem1):
    wid = lax.axis_index("s") * NUM_CORES + lax.axis_index("c")
    base_bag = wid * BAGS_PER_TILE
    idxs = (idx0, idx1)
    rows = (rows0, rows1)
    outs = (out0, out1)
    gsems = (gsem0, gsem1)
    osems = (osem0, osem1)

    def fire(ci, b):
        bag_lo = base_bag + ci * CHUNK_BAGS
        pltpu.sync_copy(ids_hbm.at[pl.ds(bag_lo, CHUNK_BAGS), :], idxs[b])
        for r in range(CHUNK_BAGS):
            pltpu.async_copy(
                table_hbm.at[idxs[b].at[r, pl.ds(0, HIST_GATHER)]],
                rows[b].at[pl.ds(r * HIST_GATHER, HIST_GATHER)], gsems[b])

    def drain_gather(b):
        pltpu.make_async_copy(table_hbm.at[pl.ds(0, ROWS_PER_CHUNK)], rows[b],
                              gsems[b]).wait()

    def drain_out(b):
        pltpu.make_async_copy(outs[b], out_hbm.at[pl.ds(0, CHUNK_BAGS)],
                              osems[b]).wait()

    pltpu.sync_copy(table_hbm.at[pl.ds(0, 1)], row0_v)
    pad_corr = tuple(
        row0_v[0, pl.ds(j * LANES, LANES)] * jnp.float32(PAD_PER_BAG)
        for j in range(VPR))

    fire(0, 0)
    fire(1, 1)

    def outer(i, carry):
        for b in range(2):
            ci = 2 * i + b
            drain_gather(b)

            @pl.when(ci >= 2)
            def _():
                drain_out(b)

            def bag_body(r, carry2):
                def red_body(k, acc):
                    row = r * HIST_GATHER + k
                    return tuple(
                        acc[j] + rows[b][row, pl.ds(j * LANES, LANES)]
                        for j in range(VPR))

                init = tuple(-pad_corr[j] for j in range(VPR))
                acc = lax.fori_loop(0, HIST_GATHER, red_body, init,
                                    unroll=8)
                for j in range(VPR):
                    outs[b][r, pl.ds(j * LANES, LANES)] = acc[j]
                return carry2

            lax.fori_loop(0, CHUNK_BAGS, bag_body, 0)
            pltpu.async_copy(
                outs[b],
                out_hbm.at[pl.ds(base_bag + ci * CHUNK_BAGS, CHUNK_BAGS)],
                osems[b])

            @pl.when(ci + 2 < N_CHUNKS)
            def _():
                fire(ci + 2, b)
        return carry

    lax.fori_loop(0, N_CHUNKS // 2, outer, 0)
    for b in range(2):
        drain_out(b)


@jax.jit
def kernel(ingredient_ids, embedding_table):
    mesh = plsc.VectorSubcoreMesh(core_axis_name="c", subcore_axis_name="s")
    f = pl.kernel(
        _sc_body,
        mesh=mesh,
        out_type=jax.ShapeDtypeStruct((BATCH, EMBED_DIM), jnp.float32),
        scratch_types=[
            pltpu.VMEM((CHUNK_BAGS, IDS_PAD), jnp.int32),
            pltpu.VMEM((CHUNK_BAGS, IDS_PAD), jnp.int32),
            pltpu.VMEM((ROWS_PER_CHUNK, EMBED_DIM), jnp.float32),
            pltpu.VMEM((ROWS_PER_CHUNK, EMBED_DIM), jnp.float32),
            pltpu.VMEM((CHUNK_BAGS, EMBED_DIM), jnp.float32),
            pltpu.VMEM((CHUNK_BAGS, EMBED_DIM), jnp.float32),
            pltpu.VMEM((1, EMBED_DIM), jnp.float32),
            pltpu.SemaphoreType.DMA,
            pltpu.SemaphoreType.DMA,
            pltpu.SemaphoreType.DMA,
            pltpu.SemaphoreType.DMA,
        ],
        compiler_params=pltpu.CompilerParams(use_tc_tiling_on_sc=False),
    )
    ids_pad = jnp.pad(ingredient_ids, ((0, 0), (0, IDS_PAD - HIST)))
    return f(ids_pad, embedding_table)


# speedup vs baseline: 10.6603x; 10.6603x over previous
import functools

import jax
import jax.numpy as jnp
from jax import lax
from jax.experimental import pallas as pl
from jax.experimental.pallas import tpu as pltpu
from jax.experimental.pallas import tpu_sc as plsc

VOCAB = 100000
EMBED_DIM = 64
BATCH = 16384
HIST = 50

NUM_CORES = 2
NUM_SUBCORES = 16
NUM_TILES = NUM_CORES * NUM_SUBCORES
LANES = 16
VPR = EMBED_DIM // LANES

BAGS_PER_TILE = BATCH // NUM_TILES
CHUNK_BAGS = 16
IDX_PER_CHUNK = CHUNK_BAGS * HIST
N_CHUNKS = BAGS_PER_TILE // CHUNK_BAGS
HIST_PAD = 56


def _sc_body(ids_hbm, table_hbm, out_hbm,
             idx0, idx1, rows0, rows1, out0, out1,
             gsem0, gsem1, osem0, osem1):
    wid = lax.axis_index("s") * NUM_CORES + lax.axis_index("c")
    base_bag = wid * BAGS_PER_TILE
    idxs = (idx0, idx1)
    rows = (rows0, rows1)
    outs = (out0, out1)
    gsems = (gsem0, gsem1)
    osems = (osem0, osem1)

    def fire(ci, b):
        bag_lo = base_bag + ci * CHUNK_BAGS
        pltpu.sync_copy(
            ids_hbm.at[pl.ds(0, HIST), pl.ds(bag_lo, CHUNK_BAGS)], idxs[b])

        def fire_k(k, carry):
            pltpu.async_copy(table_hbm.at[idxs[b].at[k]],
                             rows[b].at[pl.ds(k * CHUNK_BAGS, CHUNK_BAGS)],
                             gsems[b])
            return carry

        lax.fori_loop(0, HIST, fire_k, 0, unroll=5)

    def drain_gather(b):
        pltpu.make_async_copy(table_hbm.at[pl.ds(0, IDX_PER_CHUNK)], rows[b],
                              gsems[b]).wait()

    def drain_out(b):
        pltpu.make_async_copy(outs[b], out_hbm.at[pl.ds(0, CHUNK_BAGS)],
                              osems[b]).wait()

    fire(0, 0)
    fire(1, 1)

    def outer(i, carry):
        for b in range(2):
            ci = 2 * i + b
            drain_gather(b)

            @pl.when(ci >= 2)
            def _():
                drain_out(b)

            def bag_body(r, carry2):
                def red_body(k, acc):
                    row = k * CHUNK_BAGS + r
                    return tuple(
                        acc[j] + rows[b][row, pl.ds(j * LANES, LANES)]
                        for j in range(VPR))

                zero = jnp.zeros((LANES,), jnp.float32)
                acc = lax.fori_loop(0, HIST, red_body, (zero,) * VPR,
                                    unroll=10)
                for j in range(VPR):
                    outs[b][r, pl.ds(j * LANES, LANES)] = acc[j]
                return carry2

            lax.fori_loop(0, CHUNK_BAGS, bag_body, 0)
            pltpu.async_copy(
                outs[b],
                out_hbm.at[pl.ds(base_bag + ci * CHUNK_BAGS, CHUNK_BAGS)],
                osems[b])

            @pl.when(ci + 2 < N_CHUNKS)
            def _():
                fire(ci + 2, b)
        return carry

    lax.fori_loop(0, N_CHUNKS // 2, outer, 0)
    for b in range(2):
        drain_out(b)


@jax.jit
def kernel(ingredient_ids, embedding_table):
    ids_t = jnp.pad(ingredient_ids.T, ((0, HIST_PAD - HIST), (0, 0)))
    mesh = plsc.VectorSubcoreMesh(core_axis_name="c", subcore_axis_name="s")
    f = pl.kernel(
        _sc_body,
        mesh=mesh,
        out_type=jax.ShapeDtypeStruct((BATCH, EMBED_DIM), jnp.float32),
        scratch_types=[
            pltpu.VMEM((HIST, CHUNK_BAGS), jnp.int32),
            pltpu.VMEM((HIST, CHUNK_BAGS), jnp.int32),
            pltpu.VMEM((IDX_PER_CHUNK, EMBED_DIM), jnp.float32),
            pltpu.VMEM((IDX_PER_CHUNK, EMBED_DIM), jnp.float32),
            pltpu.VMEM((CHUNK_BAGS, EMBED_DIM), jnp.float32),
            pltpu.VMEM((CHUNK_BAGS, EMBED_DIM), jnp.float32),
            pltpu.SemaphoreType.DMA,
            pltpu.SemaphoreType.DMA,
            pltpu.SemaphoreType.DMA,
            pltpu.SemaphoreType.DMA,
        ],
        compiler_params=pltpu.CompilerParams(use_tc_tiling_on_sc=False),
    )
    return f(ids_t, embedding_table)
